# Initial kernel scaffold; baseline (speedup 1.0000x reference)
import jax
import jax.numpy as jnp
from jax.experimental import pallas as pl


def kernel(tensor):
    raise NotImplementedError("write your pallas kernel here")


# speedup vs baseline: 1.0016x; 1.0016x over previous
import jax
import jax.numpy as jnp
from jax.experimental import pallas as pl

MAX_INT = 15.0


def _fq_kernel(x_ref, o_ref):
    x = x_ref[...]
    mn = jnp.min(x, axis=-1, keepdims=True)
    mx = jnp.max(x, axis=-1, keepdims=True)
    scale = jnp.maximum((mx - mn) * (1.0 / MAX_INT), 1e-05)
    q = jnp.round(jnp.clip((x - mn) / scale, 0.0, MAX_INT))
    o_ref[...] = q * scale + mn


def kernel(tensor):
    bs, num_heads, seqlen, head_dim = tensor.shape
    rows = bs * num_heads * seqlen
    x = tensor.reshape(rows, head_dim)
    block_rows = 2048
    out = pl.pallas_call(
        _fq_kernel,
        out_shape=jax.ShapeDtypeStruct((rows, head_dim), tensor.dtype),
        grid=(rows // block_rows,),
        in_specs=[pl.BlockSpec((block_rows, head_dim), lambda i: (i, 0))],
        out_specs=pl.BlockSpec((block_rows, head_dim), lambda i: (i, 0)),
    )(x)
    return out.reshape(bs, num_heads, seqlen, head_dim)


# speedup vs baseline: 1.4621x; 1.4599x over previous
import jax
import jax.numpy as jnp
from jax.experimental import pallas as pl

MAX_INT = 15.0


def _fq_kernel(x_ref, o_ref):
    x = x_ref[...]
    mn = jnp.min(x, axis=-1, keepdims=True)
    mx = jnp.max(x, axis=-1, keepdims=True)
    scale = jnp.maximum((mx - mn) * (1.0 / MAX_INT), 1e-05)
    q = jnp.round(jnp.clip((x - mn) / scale, 0.0, MAX_INT))
    o_ref[...] = q * scale + mn


def kernel(tensor):
    bs, num_heads, seqlen, head_dim = tensor.shape
    rows = bs * num_heads * seqlen
    x = tensor.reshape(rows, head_dim)
    block_rows = 8192
    out = pl.pallas_call(
        _fq_kernel,
        out_shape=jax.ShapeDtypeStruct((rows, head_dim), tensor.dtype),
        grid=(rows // block_rows,),
        in_specs=[pl.BlockSpec((block_rows, head_dim), lambda i: (i, 0))],
        out_specs=pl.BlockSpec((block_rows, head_dim), lambda i: (i, 0)),
    )(x)
    return out.reshape(bs, num_heads, seqlen, head_dim)


# speedup vs baseline: 1.5605x; 1.0673x over previous
import jax
import jax.numpy as jnp
from jax.experimental import pallas as pl

MAX_INT = 15.0


def _fq_kernel(x_ref, o_ref):
    x = x_ref[...]
    mn = jnp.min(x, axis=-1, keepdims=True)
    mx = jnp.max(x, axis=-1, keepdims=True)
    scale = jnp.maximum((mx - mn) * (1.0 / MAX_INT), 1e-05)
    q = jnp.round(jnp.clip((x - mn) / scale, 0.0, MAX_INT))
    o_ref[...] = q * scale + mn


def kernel(tensor):
    bs, num_heads, seqlen, head_dim = tensor.shape
    rows = bs * num_heads * seqlen
    x = tensor.reshape(rows, head_dim)
    block_rows = 16384
    out = pl.pallas_call(
        _fq_kernel,
        out_shape=jax.ShapeDtypeStruct((rows, head_dim), tensor.dtype),
        grid=(rows // block_rows,),
        in_specs=[pl.BlockSpec((block_rows, head_dim), lambda i: (i, 0))],
        out_specs=pl.BlockSpec((block_rows, head_dim), lambda i: (i, 0)),
    )(x)
    return out.reshape(bs, num_heads, seqlen, head_dim)


# speedup vs baseline: 1.6162x; 1.0357x over previous
import jax
import jax.numpy as jnp
from jax.experimental import pallas as pl

MAX_INT = 15.0


def _fq_kernel(x_ref, o_ref):
    x = x_ref[...]
    mn = jnp.min(x, axis=-1, keepdims=True)
    mx = jnp.max(x, axis=-1, keepdims=True)
    scale = jnp.maximum((mx - mn) * (1.0 / MAX_INT), 1e-05)
    q = jnp.round((x - mn) / scale)
    o_ref[...] = q * scale + mn


def kernel(tensor):
    bs, num_heads, seqlen, head_dim = tensor.shape
    rows = bs * num_heads * seqlen
    x = tensor.reshape(rows, head_dim)
    block_rows = 16384
    out = pl.pallas_call(
        _fq_kernel,
        out_shape=jax.ShapeDtypeStruct((rows, head_dim), tensor.dtype),
        grid=(rows // block_rows,),
        in_specs=[pl.BlockSpec((block_rows, head_dim), lambda i: (i, 0))],
        out_specs=pl.BlockSpec((block_rows, head_dim), lambda i: (i, 0)),
    )(x)
    return out.reshape(bs, num_heads, seqlen, head_dim)
